# baseline (speedup 1.0000x reference)
import functools

import jax
import jax.numpy as jnp
from jax import lax
from jax.experimental import pallas as pl
from jax.experimental.pallas import tpu as pltpu
from jax.experimental.pallas import tpu_sc as plsc

_NC = 2
_NS = 16
_NW = _NC * _NS
_HBINS = 65536


def _key16(v):
    u = plsc.bitcast(v, jnp.uint32)
    flip = jnp.where(
        (u >> 31) != 0, jnp.uint32(0xFFFFFFFF), jnp.uint32(0x80000000)
    )
    return u ^ flip


def _zero_ref(ref, nwords):
    z = jnp.zeros((16,), jnp.float32)

    def b(i, _):
        for u in range(16):
            ref[pl.ds((i * 16 + u) * 16, 16)] = z
        return 0

    lax.fori_loop(0, nwords // 256, b, 0)


def _descending_scan(load_vreg, nvec, r):
    iota = lax.iota(jnp.int32, 16)

    def body(i, carry):
        carried, found, entry_s, above_s = carry
        t = nvec - 1 - i
        v = load_vreg(t)
        rv = lax.rev(v, (0,))
        cs = plsc.cumsum(rv) + carried
        m = cs >= r
        npos = jnp.max(plsc.all_reduce_population_count(m))
        j = jnp.max(plsc.all_reduce_ffs(m))
        entry = t * 16 + 15 - j
        prev = jnp.where(
            j == 0, carried, jnp.sum(jnp.where(iota == j - 1, cs, 0.0))
        )
        hit = (npos > 0) & (found == 0)
        entry_s = jnp.where(hit, entry, entry_s)
        above_s = jnp.where(hit, prev, above_s)
        found = jnp.where(npos > 0, 1, found)
        carried = jnp.max(cs)
        return carried, found, entry_s, above_s

    _, _, entry_s, above_s = lax.fori_loop(
        0,
        nvec,
        body,
        (jnp.float32(0.0), jnp.int32(0), jnp.int32(0), jnp.float32(0.0)),
    )
    return entry_s, above_s


def _find_bin(hist_ref, sums16_ref, coarse_ref, r):
    iota = lax.iota(jnp.int32, 16)

    def _tree_sum(vals):
        while len(vals) > 1:
            vals = [
                vals[i] + vals[i + 1] if i + 1 < len(vals) else vals[i]
                for i in range(0, len(vals), 2)
            ]
        return vals[0]

    def a_body(c, _):
        vals = [hist_ref[pl.ds(c * 256 + j * 16, 16)] for j in range(16)]
        sums16_ref[pl.ds(c * 16, 16)] = _tree_sum(vals)
        return 0

    lax.fori_loop(0, 256, a_body, 0)

    def b_body(t, _):
        base = (t * 16 + iota) * 16
        vals = [plsc.load_gather(sums16_ref, [base + j]) for j in range(16)]
        coarse_ref[pl.ds(t * 16, 16)] = _tree_sum(vals)
        return 0

    lax.fori_loop(0, 16, b_body, 0)

    c_star, above_c = _descending_scan(
        lambda t: coarse_ref[pl.ds(t * 16, 16)], 16, r
    )
    r1 = r - above_c
    f_star, above_f = _descending_scan(
        lambda t: hist_ref[pl.ds(c_star * 256 + t * 16, 16)], 16, r1
    )
    return c_star * 256 + f_star, r1 - above_f


_UNROLL = 12


def _stream_pass(xf, base, chunk, nchunk, buf_a, buf_b, sem_a, sem_b, process):
    npair = nchunk // 2

    pltpu.async_copy(xf.at[pl.ds(base, chunk)], buf_a, sem_a)

    def pair_body(p, _):
        base_p = base + p * 2 * chunk
        pltpu.make_async_copy(
            xf.at[pl.ds(base_p, chunk)], buf_a, sem_a
        ).wait()
        pltpu.async_copy(
            xf.at[pl.ds(base_p + chunk, chunk)], buf_b, sem_b
        )
        process(buf_a)
        pltpu.make_async_copy(
            xf.at[pl.ds(base_p + chunk, chunk)], buf_b, sem_b
        ).wait()

        @pl.when(p + 1 < npair)
        def _():
            pltpu.async_copy(
                xf.at[pl.ds(base_p + 2 * chunk, chunk)], buf_a, sem_a
            )

        process(buf_b)
        return 0

    lax.fori_loop(0, npair, pair_body, 0)


def _sc_body(n, k, chunk, nchunk, rows_per_w,
             xf, xo, buf_a, buf_b, obuf_a, obuf_b, hist, sums16, coarse,
             sem_a, sem_b, sem_oa, sem_ob):
    wid = lax.axis_index("s") * _NC + lax.axis_index("c")
    ones = jnp.ones((16,), jnp.float32)
    kf = jnp.float32(k)
    nvreg = chunk // 16

    for local in range(rows_per_w):
        row = wid * rows_per_w + local
        base = row * n

        _zero_ref(hist, _HBINS)

        def p1(cur):
            def vb(i, _):
                idxs = [
                    plsc.bitcast(
                        _key16(cur[pl.ds((i * _UNROLL + u) * 16, 16)]) >> 16,
                        jnp.int32,
                    )
                    for u in range(_UNROLL)
                ]
                for idx in idxs:
                    plsc.addupdate_scatter(hist, [idx], ones)
                return 0

            lax.fori_loop(0, nvreg // _UNROLL, vb, 0)

        _stream_pass(xf, base, chunk, nchunk, buf_a, buf_b, sem_a, sem_b, p1)

        b_star, r1 = _find_bin(hist, sums16, coarse, kf)
        b_star_u = b_star.astype(jnp.uint32)

        _zero_ref(hist, _HBINS)

        def p2(cur):
            def vb2(i, _):
                kus = [
                    _key16(cur[pl.ds((i * _UNROLL + u) * 16, 16)])
                    for u in range(_UNROLL)
                ]
                work = [
                    (
                        plsc.bitcast(ku & jnp.uint32(0xFFFF), jnp.int32),
                        (ku >> 16) == b_star_u,
                    )
                    for ku in kus
                ]
                for idx, msk in work:
                    plsc.addupdate_scatter(hist, [idx], ones, mask=msk)
                return 0

            lax.fori_loop(0, nvreg // _UNROLL, vb2, 0)

        _stream_pass(xf, base, chunk, nchunk, buf_a, buf_b, sem_a, sem_b, p2)

        lo_star, _ = _find_bin(hist, sums16, coarse, r1)

        k_star = (b_star_u << 16) | lo_star.astype(jnp.uint32)
        kvec = jnp.broadcast_to(k_star, (16,))
        unflip = jnp.where(
            (kvec >> 31) != 0, jnp.uint32(0x80000000), jnp.uint32(0xFFFFFFFF)
        )
        thrv = plsc.bitcast(kvec ^ unflip, jnp.float32)

        def p3(cur, ob):
            def vb3(i, _):
                vs = [
                    cur[pl.ds((i * _UNROLL + u) * 16, 16)]
                    for u in range(_UNROLL)
                ]
                outs = [jnp.where(v >= thrv, v, jnp.float32(0.0)) for v in vs]
                for u, o in enumerate(outs):
                    ob[pl.ds((i * _UNROLL + u) * 16, 16)] = o
                return 0

            lax.fori_loop(0, nvreg // _UNROLL, vb3, 0)

        npair = nchunk // 2
        pltpu.async_copy(xf.at[pl.ds(base, chunk)], buf_a, sem_a)

        def pair3(p, _):
            base_p = base + p * 2 * chunk
            pltpu.make_async_copy(
                xf.at[pl.ds(base_p, chunk)], buf_a, sem_a
            ).wait()
            pltpu.async_copy(
                xf.at[pl.ds(base_p + chunk, chunk)], buf_b, sem_b
            )

            @pl.when(p > 0)
            def _():
                pltpu.make_async_copy(
                    obuf_a, xo.at[pl.ds(base_p - 2 * chunk, chunk)], sem_oa
                ).wait()

            p3(buf_a, obuf_a)
            pltpu.async_copy(
                obuf_a, xo.at[pl.ds(base_p, chunk)], sem_oa
            )
            pltpu.make_async_copy(
                xf.at[pl.ds(base_p + chunk, chunk)], buf_b, sem_b
            ).wait()

            @pl.when(p + 1 < npair)
            def _():
                pltpu.async_copy(
                    xf.at[pl.ds(base_p + 2 * chunk, chunk)], buf_a, sem_a
                )

            @pl.when(p > 0)
            def _():
                pltpu.make_async_copy(
                    obuf_b, xo.at[pl.ds(base_p - chunk, chunk)], sem_ob
                ).wait()

            p3(buf_b, obuf_b)
            pltpu.async_copy(
                obuf_b, xo.at[pl.ds(base_p + chunk, chunk)], sem_ob
            )
            return 0

        lax.fori_loop(0, npair, pair3, 0)
        pltpu.make_async_copy(
            obuf_a, xo.at[pl.ds(base + (nchunk - 2) * chunk, chunk)], sem_oa
        ).wait()
        pltpu.make_async_copy(
            obuf_b, xo.at[pl.ds(base + (nchunk - 1) * chunk, chunk)], sem_ob
        ).wait()


def kernel(x):
    b, c, h, w = x.shape
    n = c * h * w
    k = int(0.1 * n)
    assert n % 128 == 0 and b % _NW == 0
    rows_per_w = b // _NW
    chunk = 9408
    assert n % chunk == 0
    nchunk = n // chunk
    assert nchunk % 2 == 0 and (chunk // 16) % _UNROLL == 0

    mesh = plsc.VectorSubcoreMesh(
        core_axis_name="c", subcore_axis_name="s",
        num_cores=_NC, num_subcores=_NS,
    )
    sc_fn = pl.kernel(
        functools.partial(_sc_body, n, k, chunk, nchunk, rows_per_w),
        out_type=jax.ShapeDtypeStruct((b * n,), jnp.float32),
        mesh=mesh,
        scratch_types=[
            pltpu.VMEM((chunk,), jnp.float32),
            pltpu.VMEM((chunk,), jnp.float32),
            pltpu.VMEM((chunk,), jnp.float32),
            pltpu.VMEM((chunk,), jnp.float32),
            pltpu.VMEM((_HBINS,), jnp.float32),
            pltpu.VMEM((4096,), jnp.float32),
            pltpu.VMEM((256,), jnp.float32),
            pltpu.SemaphoreType.DMA,
            pltpu.SemaphoreType.DMA,
            pltpu.SemaphoreType.DMA,
            pltpu.SemaphoreType.DMA,
        ],
        compiler_params=pltpu.CompilerParams(needs_layout_passes=False),
    )
    out = sc_fn(x.reshape(b * n))
    return out.reshape(x.shape)


# speedup vs baseline: 1.3828x; 1.3828x over previous
import functools

import jax
import jax.numpy as jnp
from jax import lax
from jax.experimental import pallas as pl
from jax.experimental.pallas import tpu as pltpu
from jax.experimental.pallas import tpu_sc as plsc

_NC = 2
_NS = 16
_NW = _NC * _NS
_HBINS = 65536


def _key16(v):
    u = plsc.bitcast(v, jnp.uint32)
    flip = jnp.where(
        (u >> 31) != 0, jnp.uint32(0xFFFFFFFF), jnp.uint32(0x80000000)
    )
    return u ^ flip


def _zero_ref(ref, nwords):
    z = jnp.zeros((16,), jnp.float32)

    def b(i, _):
        for u in range(16):
            ref[pl.ds((i * 16 + u) * 16, 16)] = z
        return 0

    lax.fori_loop(0, nwords // 256, b, 0)


def _descending_scan(load_vreg, nvec, r):
    iota = lax.iota(jnp.int32, 16)

    def body(i, carry):
        carried, found, entry_s, above_s = carry
        t = nvec - 1 - i
        v = load_vreg(t)
        rv = lax.rev(v, (0,))
        cs = plsc.cumsum(rv) + carried
        m = cs >= r
        npos = jnp.max(plsc.all_reduce_population_count(m))
        j = jnp.max(plsc.all_reduce_ffs(m))
        entry = t * 16 + 15 - j
        prev = jnp.where(
            j == 0, carried, jnp.sum(jnp.where(iota == j - 1, cs, 0.0))
        )
        hit = (npos > 0) & (found == 0)
        entry_s = jnp.where(hit, entry, entry_s)
        above_s = jnp.where(hit, prev, above_s)
        found = jnp.where(npos > 0, 1, found)
        carried = jnp.max(cs)
        return carried, found, entry_s, above_s

    _, _, entry_s, above_s = lax.fori_loop(
        0,
        nvec,
        body,
        (jnp.float32(0.0), jnp.int32(0), jnp.int32(0), jnp.float32(0.0)),
    )
    return entry_s, above_s


def _find_bin(hist_ref, sums16_ref, coarse_ref, r):
    iota = lax.iota(jnp.int32, 16)

    def _tree_sum(vals):
        while len(vals) > 1:
            vals = [
                vals[i] + vals[i + 1] if i + 1 < len(vals) else vals[i]
                for i in range(0, len(vals), 2)
            ]
        return vals[0]

    def a_body(c, _):
        vals = [hist_ref[pl.ds(c * 256 + j * 16, 16)] for j in range(16)]
        sums16_ref[pl.ds(c * 16, 16)] = _tree_sum(vals)
        return 0

    lax.fori_loop(0, 256, a_body, 0)

    def b_body(t, _):
        base = (t * 16 + iota) * 16
        vals = [plsc.load_gather(sums16_ref, [base + j]) for j in range(16)]
        coarse_ref[pl.ds(t * 16, 16)] = _tree_sum(vals)
        return 0

    lax.fori_loop(0, 16, b_body, 0)

    c_star, above_c = _descending_scan(
        lambda t: coarse_ref[pl.ds(t * 16, 16)], 16, r
    )
    r1 = r - above_c
    f_star, above_f = _descending_scan(
        lambda t: hist_ref[pl.ds(c_star * 256 + t * 16, 16)], 16, r1
    )
    return c_star * 256 + f_star, r1 - above_f


_UNROLL = 12


def _stream_pass(xf, base, chunk, nchunk, buf_a, buf_b, sem_a, sem_b, process):
    npair = nchunk // 2

    pltpu.async_copy(xf.at[pl.ds(base, chunk)], buf_a, sem_a)

    def pair_body(p, _):
        base_p = base + p * 2 * chunk
        pltpu.make_async_copy(
            xf.at[pl.ds(base_p, chunk)], buf_a, sem_a
        ).wait()
        pltpu.async_copy(
            xf.at[pl.ds(base_p + chunk, chunk)], buf_b, sem_b
        )
        process(buf_a)
        pltpu.make_async_copy(
            xf.at[pl.ds(base_p + chunk, chunk)], buf_b, sem_b
        ).wait()

        @pl.when(p + 1 < npair)
        def _():
            pltpu.async_copy(
                xf.at[pl.ds(base_p + 2 * chunk, chunk)], buf_a, sem_a
            )

        process(buf_b)
        return 0

    lax.fori_loop(0, npair, pair_body, 0)


def _sc_body(n, k, chunk, nchunk, rows_per_w,
             xf, out, buf_a, buf_b, hist, sums16, coarse, tout, sem_a, sem_b):
    wid = lax.axis_index("s") * _NC + lax.axis_index("c")
    iota = lax.iota(jnp.int32, 16)
    ones = jnp.ones((16,), jnp.float32)
    kf = jnp.float32(k)
    nvreg = chunk // 16

    tvec = jnp.zeros((16,), jnp.float32)
    for local in range(rows_per_w):
        row = wid * rows_per_w + local
        base = row * n

        _zero_ref(hist, _HBINS)

        def p1(cur):
            def vb(i, _):
                idxs = [
                    plsc.bitcast(
                        _key16(cur[pl.ds((i * _UNROLL + u) * 16, 16)]) >> 16,
                        jnp.int32,
                    )
                    for u in range(_UNROLL)
                ]
                for idx in idxs:
                    plsc.addupdate_scatter(hist, [idx], ones)
                return 0

            lax.fori_loop(0, nvreg // _UNROLL, vb, 0)

        _stream_pass(xf, base, chunk, nchunk, buf_a, buf_b, sem_a, sem_b, p1)

        b_star, r1 = _find_bin(hist, sums16, coarse, kf)
        b_star_u = b_star.astype(jnp.uint32)

        _zero_ref(hist, _HBINS)

        def p2(cur):
            def vb2(i, _):
                kus = [
                    _key16(cur[pl.ds((i * _UNROLL + u) * 16, 16)])
                    for u in range(_UNROLL)
                ]
                work = [
                    (
                        plsc.bitcast(ku & jnp.uint32(0xFFFF), jnp.int32),
                        (ku >> 16) == b_star_u,
                    )
                    for ku in kus
                ]
                for idx, msk in work:
                    plsc.addupdate_scatter(hist, [idx], ones, mask=msk)
                return 0

            lax.fori_loop(0, nvreg // _UNROLL, vb2, 0)

        _stream_pass(xf, base, chunk, nchunk, buf_a, buf_b, sem_a, sem_b, p2)

        lo_star, _ = _find_bin(hist, sums16, coarse, r1)

        k_star = (b_star_u << 16) | lo_star.astype(jnp.uint32)
        kvec = jnp.broadcast_to(k_star, (16,))
        unflip = jnp.where(
            (kvec >> 31) != 0, jnp.uint32(0x80000000), jnp.uint32(0xFFFFFFFF)
        )
        thr = plsc.bitcast(kvec ^ unflip, jnp.float32)
        tvec = jnp.where(iota == local, thr, tvec)

    tout[...] = tvec
    pltpu.sync_copy(tout, out.at[wid])


def _mask_body(x_ref, t_ref, o_ref):
    xv = x_ref[...]
    tval = t_ref[0, 0, 0]
    o_ref[...] = xv * (xv >= tval).astype(xv.dtype)


def kernel(x):
    b, c, h, w = x.shape
    n = c * h * w
    k = int(0.1 * n)
    assert n % 128 == 0 and b % _NW == 0
    rows_per_w = b // _NW
    chunk = 9408
    assert n % chunk == 0
    nchunk = n // chunk
    assert nchunk % 2 == 0 and (chunk // 16) % _UNROLL == 0

    mesh = plsc.VectorSubcoreMesh(
        core_axis_name="c", subcore_axis_name="s",
        num_cores=_NC, num_subcores=_NS,
    )
    sc_fn = pl.kernel(
        functools.partial(_sc_body, n, k, chunk, nchunk, rows_per_w),
        out_type=jax.ShapeDtypeStruct((_NW, 16), jnp.float32),
        mesh=mesh,
        scratch_types=[
            pltpu.VMEM((chunk,), jnp.float32),
            pltpu.VMEM((chunk,), jnp.float32),
            pltpu.VMEM((_HBINS,), jnp.float32),
            pltpu.VMEM((4096,), jnp.float32),
            pltpu.VMEM((256,), jnp.float32),
            pltpu.VMEM((16,), jnp.float32),
            pltpu.SemaphoreType.DMA,
            pltpu.SemaphoreType.DMA,
        ],
        compiler_params=pltpu.CompilerParams(needs_layout_passes=False),
    )
    thr2d = sc_fn(x.reshape(b * n))
    thr = thr2d[:, :rows_per_w].reshape(b)

    x3 = x.reshape(b, c, h * w)
    thrb = jnp.broadcast_to(thr[:, None, None], (b, 1, 128))
    out = pl.pallas_call(
        _mask_body,
        grid=(b,),
        in_specs=[
            pl.BlockSpec((1, c, h * w), lambda i: (i, 0, 0)),
            pl.BlockSpec((1, 1, 128), lambda i: (i, 0, 0)),
        ],
        out_specs=pl.BlockSpec((1, c, h * w), lambda i: (i, 0, 0)),
        out_shape=jax.ShapeDtypeStruct((b, c, h * w), jnp.float32),
    )(x3, thrb)
    return out.reshape(x.shape)


# speedup vs baseline: 1.8505x; 1.3382x over previous
import functools

import jax
import jax.numpy as jnp
from jax import lax
from jax.experimental import pallas as pl
from jax.experimental.pallas import tpu as pltpu
from jax.experimental.pallas import tpu_sc as plsc

_NC = 2
_NS = 16
_NW = _NC * _NS
_HBINS = 65536
_UNROLL = 14
_CPC = 8


def _key16(v):
    u = plsc.bitcast(v, jnp.uint32)
    flip = jnp.where(
        (u >> 31) != 0, jnp.uint32(0xFFFFFFFF), jnp.uint32(0x80000000)
    )
    return u ^ flip


def _zero_ref(ref, nwords):
    z = jnp.zeros((16,), jnp.float32)

    def b(i, _):
        for u in range(16):
            ref[pl.ds((i * 16 + u) * 16, 16)] = z
        return 0

    lax.fori_loop(0, nwords // 256, b, 0)


def _descending_scan(load_vreg, nvec, r):
    iota = lax.iota(jnp.int32, 16)

    def body(i, carry):
        carried, found, entry_s, above_s = carry
        t = nvec - 1 - i
        v = load_vreg(t)
        rv = lax.rev(v, (0,))
        cs = plsc.cumsum(rv) + carried
        m = cs >= r
        npos = jnp.max(plsc.all_reduce_population_count(m))
        j = jnp.max(plsc.all_reduce_ffs(m))
        entry = t * 16 + 15 - j
        prev = jnp.where(
            j == 0, carried, jnp.sum(jnp.where(iota == j - 1, cs, 0.0))
        )
        hit = (npos > 0) & (found == 0)
        entry_s = jnp.where(hit, entry, entry_s)
        above_s = jnp.where(hit, prev, above_s)
        found = jnp.where(npos > 0, 1, found)
        carried = jnp.max(cs)
        return carried, found, entry_s, above_s

    _, _, entry_s, above_s = lax.fori_loop(
        0,
        nvec,
        body,
        (jnp.float32(0.0), jnp.int32(0), jnp.int32(0), jnp.float32(0.0)),
    )
    return entry_s, above_s


def _find_bin(hist_ref, sums16_ref, coarse_ref, r):
    iota = lax.iota(jnp.int32, 16)

    def _tree_sum(vals):
        while len(vals) > 1:
            vals = [
                vals[i] + vals[i + 1] if i + 1 < len(vals) else vals[i]
                for i in range(0, len(vals), 2)
            ]
        return vals[0]

    def a_body(c, _):
        vals = [hist_ref[pl.ds(c * 256 + j * 16, 16)] for j in range(16)]
        sums16_ref[pl.ds(c * 16, 16)] = _tree_sum(vals)
        return 0

    lax.fori_loop(0, 256, a_body, 0)

    def b_body(t, _):
        base = (t * 16 + iota) * 16
        vals = [plsc.load_gather(sums16_ref, [base + j]) for j in range(16)]
        coarse_ref[pl.ds(t * 16, 16)] = _tree_sum(vals)
        return 0

    lax.fori_loop(0, 16, b_body, 0)

    c_star, above_c = _descending_scan(
        lambda t: coarse_ref[pl.ds(t * 16, 16)], 16, r
    )
    r1 = r - above_c
    f_star, above_f = _descending_scan(
        lambda t: hist_ref[pl.ds(c_star * 256 + t * 16, 16)], 16, r1
    )
    return c_star * 256 + f_star, r1 - above_f


def _stream_pass(x3, row, nchunk, buf_a, buf_b, sem_a, sem_b, process):
    npair = nchunk // 2

    pltpu.async_copy(x3.at[row, pl.ds(0, _CPC)], buf_a, sem_a)

    def pair_body(p, _):
        c_p = pl.multiple_of(p * 2 * _CPC, _CPC)
        pltpu.make_async_copy(
            x3.at[row, pl.ds(c_p, _CPC)], buf_a, sem_a
        ).wait()
        pltpu.async_copy(
            x3.at[row, pl.ds(c_p + _CPC, _CPC)], buf_b, sem_b
        )
        process(buf_a)
        pltpu.make_async_copy(
            x3.at[row, pl.ds(c_p + _CPC, _CPC)], buf_b, sem_b
        ).wait()

        @pl.when(p + 1 < npair)
        def _():
            pltpu.async_copy(
                x3.at[row, pl.ds(c_p + 2 * _CPC, _CPC)], buf_a, sem_a
            )

        process(buf_b)
        return 0

    lax.fori_loop(0, npair, pair_body, 0)


def _sc_body(hw, k, nchunk, rows_per_w,
             x3, xo, buf_a, buf_b, hist, sums16, coarse,
             sem_a, sem_b, sem_ob):
    wid = lax.axis_index("s") * _NC + lax.axis_index("c")
    ones = jnp.ones((16,), jnp.float32)
    kf = jnp.float32(k)
    nvr = hw // 16

    for local in range(rows_per_w):
        row = wid * rows_per_w + local

        _zero_ref(hist, _HBINS)

        def p1(cur):
            def cb(ci, _):
                def vb(i, _):
                    idxs = [
                        plsc.bitcast(
                            _key16(cur[ci, pl.ds((i * _UNROLL + u) * 16, 16)])
                            >> 16,
                            jnp.int32,
                        )
                        for u in range(_UNROLL)
                    ]
                    for idx in idxs:
                        plsc.addupdate_scatter(hist, [idx], ones)
                    return 0

                lax.fori_loop(0, nvr // _UNROLL, vb, 0)
                return 0

            lax.fori_loop(0, _CPC, cb, 0)

        _stream_pass(x3, row, nchunk, buf_a, buf_b, sem_a, sem_b, p1)

        b_star, r1 = _find_bin(hist, sums16, coarse, kf)
        b_star_u = b_star.astype(jnp.uint32)

        _zero_ref(hist, _HBINS)

        def p2(cur):
            def cb2(ci, _):
                def vb2(i, _):
                    kus = [
                        _key16(cur[ci, pl.ds((i * _UNROLL + u) * 16, 16)])
                        for u in range(_UNROLL)
                    ]
                    work = [
                        (
                            plsc.bitcast(ku & jnp.uint32(0xFFFF), jnp.int32),
                            (ku >> 16) == b_star_u,
                        )
                        for ku in kus
                    ]
                    for idx, msk in work:
                        plsc.addupdate_scatter(hist, [idx], ones, mask=msk)
                    return 0

                lax.fori_loop(0, nvr // _UNROLL, vb2, 0)
                return 0

            lax.fori_loop(0, _CPC, cb2, 0)

        _stream_pass(x3, row, nchunk, buf_a, buf_b, sem_a, sem_b, p2)

        lo_star, _ = _find_bin(hist, sums16, coarse, r1)

        k_star = (b_star_u << 16) | lo_star.astype(jnp.uint32)
        kvec = jnp.broadcast_to(k_star, (16,))
        unflip = jnp.where(
            (kvec >> 31) != 0, jnp.uint32(0x80000000), jnp.uint32(0xFFFFFFFF)
        )
        thrv = plsc.bitcast(kvec ^ unflip, jnp.float32)

        def p3(cur, ob):
            def cb3(ci, _):
                def vb3(i, _):
                    vs = [
                        cur[ci, pl.ds((i * _UNROLL + u) * 16, 16)]
                        for u in range(_UNROLL)
                    ]
                    outs = [
                        jnp.where(v >= thrv, v, jnp.float32(0.0)) for v in vs
                    ]
                    for u, o in enumerate(outs):
                        ob[ci, pl.ds((i * _UNROLL + u) * 16, 16)] = o
                    return 0

                lax.fori_loop(0, nvr // _UNROLL, vb3, 0)
                return 0

            lax.fori_loop(0, _CPC, cb3, 0)

        pltpu.async_copy(x3.at[row, pl.ds(0, _CPC)], buf_a, sem_a)

        def c3_body(ci, _):
            c_p = pl.multiple_of(ci * _CPC, _CPC)
            pltpu.make_async_copy(
                x3.at[row, pl.ds(c_p, _CPC)], buf_a, sem_a
            ).wait()

            @pl.when(ci > 0)
            def _():
                pltpu.make_async_copy(
                    buf_b, xo.at[row, pl.ds(c_p - _CPC, _CPC)], sem_ob
                ).wait()

            p3(buf_a, buf_b)
            pltpu.async_copy(buf_b, xo.at[row, pl.ds(c_p, _CPC)], sem_ob)

            @pl.when(ci + 1 < nchunk)
            def _():
                pltpu.async_copy(
                    x3.at[row, pl.ds(c_p + _CPC, _CPC)], buf_a, sem_a
                )

            return 0

        lax.fori_loop(0, nchunk, c3_body, 0)
        pltpu.make_async_copy(
            buf_b, xo.at[row, pl.ds((nchunk - 1) * _CPC, _CPC)], sem_ob
        ).wait()


def kernel(x):
    b, c, h, w = x.shape
    n = c * h * w
    hw = h * w
    k = int(0.1 * n)
    assert hw % (16 * _UNROLL) == 0 and b % _NW == 0
    rows_per_w = b // _NW
    assert c % (2 * _CPC) == 0
    nchunk = c // _CPC

    mesh = plsc.VectorSubcoreMesh(
        core_axis_name="c", subcore_axis_name="s",
        num_cores=_NC, num_subcores=_NS,
    )
    sc_fn = pl.kernel(
        functools.partial(_sc_body, hw, k, nchunk, rows_per_w),
        out_type=jax.ShapeDtypeStruct((b, c, hw), jnp.float32),
        mesh=mesh,
        scratch_types=[
            pltpu.VMEM((_CPC, hw), jnp.float32),
            pltpu.VMEM((_CPC, hw), jnp.float32),
            pltpu.VMEM((_HBINS,), jnp.float32),
            pltpu.VMEM((4096,), jnp.float32),
            pltpu.VMEM((256,), jnp.float32),
            pltpu.SemaphoreType.DMA,
            pltpu.SemaphoreType.DMA,
            pltpu.SemaphoreType.DMA,
        ],
        compiler_params=pltpu.CompilerParams(needs_layout_passes=False),
    )
    out = sc_fn(x.reshape(b, c, hw))
    return out.reshape(x.shape)
